# baseline (speedup 1.0000x reference)
import functools

import jax
import jax.numpy as jnp
from jax import lax
from jax.experimental import pallas as pl
from jax.experimental.pallas import tpu as pltpu
from jax.experimental.pallas import tpu_sc as plsc

L = 200
B = 4096
H = 64
VP = 500000
HP = 128
NC, NS = 2, 16
NW = NC * NS
BPW = B // NW
CG = BPW // 16
S0, S1 = 96, 104
UNROLL = 8


NSLAB = 3906
VTAIL = 999936


def _sc_pack_table(tableT, tail):
    mesh = plsc.VectorSubcoreMesh(core_axis_name="c", subcore_axis_name="s")

    @functools.partial(
        pl.kernel,
        mesh=mesh,
        out_type=jax.ShapeDtypeStruct((VP, HP), jnp.float32),
        compiler_params=pltpu.CompilerParams(
            use_tc_tiling_on_sc=True, needs_layout_passes=False),
        scratch_types=[
            pltpu.VMEM((H, 256), jnp.float32),
            pltpu.VMEM((H, 256), jnp.float32),
            pltpu.VMEM((128, HP), jnp.float32),
            pltpu.VMEM((128, HP), jnp.float32),
            pltpu.VMEM((64, H), jnp.float32),
            pltpu.SemaphoreType.DMA,
            pltpu.SemaphoreType.DMA,
            pltpu.SemaphoreType.DMA,
            pltpu.SemaphoreType.DMA,
        ],
    )
    def k(tab_hbm, tail_hbm, out_hbm, in0, in1, ob0, ob1, tbuf,
          si0, si1, so0, so1):
        wid = lax.axis_index("s") * NC + lax.axis_index("c")
        iota = lax.iota(jnp.int32, 16)
        row_base = [
            lax.shift_right_logical(iota + 16 * j, 1) for j in range(16)
        ]
        col_par = (iota & 1) * H

        def in_copy(sid, buf, sem):
            return pltpu.make_async_copy(
                tab_hbm.at[:, pl.ds(sid * 256, 256)], buf, sem)

        def out_copy(sid, buf, sem):
            return pltpu.make_async_copy(
                buf, out_hbm.at[pl.ds(sid * 128, 128)], sem)

        def transpose(src, dst, ngroups=16):
            def hbody(h, _):
                cv = col_par + jnp.broadcast_to(h, (16,))
                for j in range(ngroups):
                    v = src[h, pl.ds(16 * j, 16)]
                    plsc.store_scatter(dst, [row_base[j], cv], v)
                return 0

            lax.fori_loop(0, H, hbody, 0)

        def sid_of(t):
            return wid + NW * t

        nt = NSLAB // NW + 1

        @pl.when(sid_of(0) < NSLAB)
        def _():
            in_copy(sid_of(0), in0, si0).start()

        @pl.when(sid_of(1) < NSLAB)
        def _():
            in_copy(sid_of(1), in1, si1).start()

        def outer(t, _):
            for phase, (ib, ob, si, so) in enumerate(
                    ((in0, ob0, si0, so0), (in1, ob1, si1, so1))):
                tt = 2 * t + phase
                sid = sid_of(tt)

                @pl.when(sid < NSLAB)
                def _():
                    in_copy(sid, ib, si).wait()

                    @pl.when(tt >= 2)
                    def _():
                        out_copy(sid_of(tt - 2), ob, so).wait()

                    transpose(ib, ob)
                    out_copy(sid, ob, so).start()
                    nsid = sid_of(tt + 2)

                    @pl.when(nsid < NSLAB)
                    def _():
                        in_copy(nsid, ib, si).start()
            return 0

        lax.fori_loop(0, (nt + 1) // 2, outer, 0)

        for tt in (nt - 3, nt - 2, nt - 1):
            ob, so = (ob0, so0) if tt % 2 == 0 else (ob1, so1)

            @pl.when((sid_of(tt) < NSLAB) & (sid_of(tt + 2) >= NSLAB))
            def _():
                out_copy(sid_of(tt), ob, so).wait()

        @pl.when(wid == NW - 1)
        def _():
            pltpu.sync_copy(tail_hbm, tbuf)

            def rbody(r, _):
                for c in range(8):
                    ob0[r, pl.ds(16 * c, 16)] = tbuf[
                        2 * r + c // 4, pl.ds((c % 4) * 16, 16)]
                return 0

            lax.fori_loop(0, 32, rbody, 0)
            pltpu.sync_copy(ob0.at[pl.ds(0, 32)],
                            out_hbm.at[pl.ds(VTAIL // 2, 32)])

    return k(tableT, tail)


def _sc_embed_sum(features, table2):
    mesh = plsc.VectorSubcoreMesh(core_axis_name="c", subcore_axis_name="s")

    @functools.partial(
        pl.kernel,
        mesh=mesh,
        out_type=jax.ShapeDtypeStruct((B, H), jnp.float32),
        compiler_params=pltpu.CompilerParams(
            use_tc_tiling_on_sc=True, needs_layout_passes=False),
        scratch_types=[
            pltpu.VMEM((L, BPW), jnp.int32),
            pltpu.VMEM((BPW * L,), jnp.int32),
            pltpu.VMEM((BPW,), jnp.int32),
            pltpu.VMEM((L, HP), jnp.float32),
            pltpu.VMEM((L, HP), jnp.float32),
            pltpu.VMEM((BPW, H), jnp.float32),
            pltpu.SemaphoreType.DMA,
            pltpu.SemaphoreType.DMA,
        ],
    )
    def k(feat_hbm, table_hbm, out_hbm, idx2d, idx_v, n0_v, rows0, rows1,
          out_v, sem0, sem1):
        wid = lax.axis_index("s") * NC + lax.axis_index("c")
        base = wid * BPW

        pltpu.sync_copy(feat_hbm.at[:, pl.ds(base, BPW)], idx2d)

        lane_base = lax.iota(jnp.int32, 16) * L
        zero16 = jnp.zeros((16,), jnp.int32)
        one16 = jnp.ones((16,), jnp.int32)

        def tbody(l, carry):
            los, his = carry
            nlos, nhis = [], []
            for c in range(CG):
                v = idx2d[l, pl.ds(c * 16, 16)]
                par = v & 1
                v2 = lax.shift_right_logical(v, 1)
                is_hi = par == 1
                pos = lane_base + c * 16 * L + jnp.where(
                    is_hi, (L - 1) - his[c], los[c])
                plsc.store_scatter(idx_v, [pos], v2)
                nlos.append(los[c] + (one16 - par))
                nhis.append(his[c] + par)
            return tuple(nlos), tuple(nhis)

        los, his = lax.fori_loop(
            0, L, tbody, ((zero16,) * CG, (zero16,) * CG))
        for c in range(CG):
            n0_v[pl.ds(c * 16, 16)] = los[c]

        def copies(b, rows, sem):
            off = b * L
            c1 = pltpu.make_async_copy(
                table_hbm.at[idx_v.at[pl.ds(off, S0)]],
                rows.at[pl.ds(0, S0)], sem)
            c2 = pltpu.make_async_copy(
                table_hbm.at[idx_v.at[pl.ds(off + S0, S1)]],
                rows.at[pl.ds(S0, S1)], sem)
            return c1, c2

        def fire(b, rows, sem):
            c1, c2 = copies(b, rows, sem)
            c1.start()
            c2.start()

        def drain(b, rows, sem):
            c1, c2 = copies(b, rows, sem)
            c1.wait()
            c2.wait()

        def accumulate(b, rows):
            zero = jnp.zeros((16,), jnp.float32)
            n0b = plsc.load_gather(n0_v, [jnp.broadcast_to(b, (16,))])

            def body(j, accs):
                a0, a1, a2, a3 = accs
                for u in range(UNROLL):
                    jj = j * UNROLL + u
                    lo = jnp.broadcast_to(jj, (16,)) < n0b
                    r0 = jnp.where(lo, rows[jj, pl.ds(0, 16)],
                                   rows[jj, pl.ds(64, 16)])
                    r1 = jnp.where(lo, rows[jj, pl.ds(16, 16)],
                                   rows[jj, pl.ds(80, 16)])
                    r2 = jnp.where(lo, rows[jj, pl.ds(32, 16)],
                                   rows[jj, pl.ds(96, 16)])
                    r3 = jnp.where(lo, rows[jj, pl.ds(48, 16)],
                                   rows[jj, pl.ds(112, 16)])
                    a0, a1, a2, a3 = a0 + r0, a1 + r1, a2 + r2, a3 + r3
                return a0, a1, a2, a3

            return lax.fori_loop(0, L // UNROLL, body, (zero,) * 4)

        fire(0, rows0, sem0)
        fire(1, rows1, sem1)

        def outer(i, _):
            for phase, (rows, sem) in enumerate(((rows0, sem0), (rows1, sem1))):
                b = 2 * i + phase
                drain(b, rows, sem)
                a0, a1, a2, a3 = accumulate(b, rows)
                nb = b + 2

                @pl.when(nb < BPW)
                def _():
                    fire(nb, rows, sem)

                out_v[b, pl.ds(0, 16)] = a0
                out_v[b, pl.ds(16, 16)] = a1
                out_v[b, pl.ds(32, 16)] = a2
                out_v[b, pl.ds(48, 16)] = a3
            return 0

        lax.fori_loop(0, BPW // 2, outer, 0)
        pltpu.sync_copy(out_v, out_hbm.at[pl.ds(base, BPW)])

    return k(features, table2)


def _tc_head(h, W, b2):
    def body(h_ref, w_ref, b_ref, o_ref):
        hv = jnp.maximum(h_ref[...], 0.0)
        o_ref[...] = lax.dot_general(
            hv, w_ref[...], (((1,), (1,)), ((), ())),
            preferred_element_type=jnp.float32) + b_ref[...]

    return pl.pallas_call(
        body,
        out_shape=jax.ShapeDtypeStruct((B, H), jnp.float32),
    )(h, W, b2)


def kernel(features, table, W, b):
    table2 = _sc_pack_table(jnp.transpose(table), table[VTAIL:, :])
    pooled = _sc_embed_sum(features, table2)
    return _tc_head(pooled, W, b.reshape(1, H))


# speedup vs baseline: 1.9827x; 1.9827x over previous
import functools

import jax
import jax.numpy as jnp
from jax import lax
from jax.experimental import pallas as pl
from jax.experimental.pallas import tpu as pltpu
from jax.experimental.pallas import tpu_sc as plsc

L = 200
B = 4096
H = 64
VP = 500000
HP = 128
NC, NS = 2, 16
NW = NC * NS
BPW = B // NW
CG = BPW // 16
S0, S1 = 96, 104
UNROLL = 8


NSLAB = 3906
VTAIL = 999936


def _sc_pack_table(tableT, tail):
    mesh = plsc.VectorSubcoreMesh(core_axis_name="c", subcore_axis_name="s")

    @functools.partial(
        pl.kernel,
        mesh=mesh,
        out_type=jax.ShapeDtypeStruct((VP, HP), jnp.float32),
        compiler_params=pltpu.CompilerParams(
            use_tc_tiling_on_sc=True, needs_layout_passes=False),
        scratch_types=[
            pltpu.VMEM((H, 256), jnp.float32),
            pltpu.VMEM((H, 256), jnp.float32),
            pltpu.VMEM((128, HP), jnp.float32),
            pltpu.VMEM((128, HP), jnp.float32),
            pltpu.VMEM((64, H), jnp.float32),
            pltpu.SemaphoreType.DMA,
            pltpu.SemaphoreType.DMA,
            pltpu.SemaphoreType.DMA,
            pltpu.SemaphoreType.DMA,
        ],
    )
    def k(tab_hbm, tail_hbm, out_hbm, in0, in1, ob0, ob1, tbuf,
          si0, si1, so0, so1):
        wid = lax.axis_index("s") * NC + lax.axis_index("c")
        iota = lax.iota(jnp.int32, 16)
        hgs = [16 * g + iota for g in range(4)]

        def in_copy(sid, buf, sem):
            return pltpu.make_async_copy(
                tab_hbm.at[:, pl.ds(sid * 256, 256)], buf, sem)

        def out_copy(sid, buf, sem):
            return pltpu.make_async_copy(
                buf, out_hbm.at[pl.ds(sid * 128, 128)], sem)

        def transpose(src, dst):
            def vbody(v0, _):
                vm = v0 + iota
                vm = jnp.where(vm >= 256, vm - 256, vm)
                r = lax.shift_right_logical(vm, 1)
                p64 = (vm & 1) * H
                for g in range(4):
                    val = plsc.load_gather(src, [hgs[g], vm])
                    plsc.store_scatter(dst, [r, p64 + hgs[g]], val)
                return 0

            lax.fori_loop(0, 256, vbody, 0)

        def sid_of(t):
            return wid + NW * t

        nt = NSLAB // NW + 1

        @pl.when(sid_of(0) < NSLAB)
        def _():
            in_copy(sid_of(0), in0, si0).start()

        @pl.when(sid_of(1) < NSLAB)
        def _():
            in_copy(sid_of(1), in1, si1).start()

        def outer(t, _):
            for phase, (ib, ob, si, so) in enumerate(
                    ((in0, ob0, si0, so0), (in1, ob1, si1, so1))):
                tt = 2 * t + phase
                sid = sid_of(tt)

                @pl.when(sid < NSLAB)
                def _():
                    in_copy(sid, ib, si).wait()

                    @pl.when(tt >= 2)
                    def _():
                        out_copy(sid_of(tt - 2), ob, so).wait()

                    transpose(ib, ob)
                    out_copy(sid, ob, so).start()
                    nsid = sid_of(tt + 2)

                    @pl.when(nsid < NSLAB)
                    def _():
                        in_copy(nsid, ib, si).start()
            return 0

        lax.fori_loop(0, (nt + 1) // 2, outer, 0)

        for tt in (nt - 3, nt - 2, nt - 1):
            ob, so = (ob0, so0) if tt % 2 == 0 else (ob1, so1)

            @pl.when((sid_of(tt) < NSLAB) & (sid_of(tt + 2) >= NSLAB))
            def _():
                out_copy(sid_of(tt), ob, so).wait()

        @pl.when(wid == NW - 1)
        def _():
            pltpu.sync_copy(tail_hbm, tbuf)

            def rbody(r, _):
                for c in range(8):
                    ob0[r, pl.ds(16 * c, 16)] = tbuf[
                        2 * r + c // 4, pl.ds((c % 4) * 16, 16)]
                return 0

            lax.fori_loop(0, 32, rbody, 0)
            pltpu.sync_copy(ob0.at[pl.ds(0, 32)],
                            out_hbm.at[pl.ds(VTAIL // 2, 32)])

    return k(tableT, tail)


def _sc_embed_sum(features, table2):
    mesh = plsc.VectorSubcoreMesh(core_axis_name="c", subcore_axis_name="s")

    @functools.partial(
        pl.kernel,
        mesh=mesh,
        out_type=jax.ShapeDtypeStruct((B, H), jnp.float32),
        compiler_params=pltpu.CompilerParams(
            use_tc_tiling_on_sc=True, needs_layout_passes=False),
        scratch_types=[
            pltpu.VMEM((L, BPW), jnp.int32),
            pltpu.VMEM((BPW * L,), jnp.int32),
            pltpu.VMEM((BPW,), jnp.int32),
            pltpu.VMEM((L, HP), jnp.float32),
            pltpu.VMEM((L, HP), jnp.float32),
            pltpu.VMEM((BPW, H), jnp.float32),
            pltpu.SemaphoreType.DMA,
            pltpu.SemaphoreType.DMA,
        ],
    )
    def k(feat_hbm, table_hbm, out_hbm, idx2d, idx_v, n0_v, rows0, rows1,
          out_v, sem0, sem1):
        wid = lax.axis_index("s") * NC + lax.axis_index("c")
        base = wid * BPW

        pltpu.sync_copy(feat_hbm.at[:, pl.ds(base, BPW)], idx2d)

        lane_base = lax.iota(jnp.int32, 16) * L
        zero16 = jnp.zeros((16,), jnp.int32)
        one16 = jnp.ones((16,), jnp.int32)

        def tbody(l, carry):
            los, his = carry
            nlos, nhis = [], []
            for c in range(CG):
                v = idx2d[l, pl.ds(c * 16, 16)]
                par = v & 1
                v2 = lax.shift_right_logical(v, 1)
                is_hi = par == 1
                pos = lane_base + c * 16 * L + jnp.where(
                    is_hi, (L - 1) - his[c], los[c])
                plsc.store_scatter(idx_v, [pos], v2)
                nlos.append(los[c] + (one16 - par))
                nhis.append(his[c] + par)
            return tuple(nlos), tuple(nhis)

        los, his = lax.fori_loop(
            0, L, tbody, ((zero16,) * CG, (zero16,) * CG))
        for c in range(CG):
            n0_v[pl.ds(c * 16, 16)] = los[c]

        def copies(b, rows, sem):
            off = b * L
            c1 = pltpu.make_async_copy(
                table_hbm.at[idx_v.at[pl.ds(off, S0)]],
                rows.at[pl.ds(0, S0)], sem)
            c2 = pltpu.make_async_copy(
                table_hbm.at[idx_v.at[pl.ds(off + S0, S1)]],
                rows.at[pl.ds(S0, S1)], sem)
            return c1, c2

        def fire(b, rows, sem):
            c1, c2 = copies(b, rows, sem)
            c1.start()
            c2.start()

        def drain(b, rows, sem):
            c1, c2 = copies(b, rows, sem)
            c1.wait()
            c2.wait()

        def accumulate(b, rows):
            zero = jnp.zeros((16,), jnp.float32)
            n0b = plsc.load_gather(n0_v, [jnp.broadcast_to(b, (16,))])

            def body(j, accs):
                a0, a1, a2, a3 = accs
                for u in range(UNROLL):
                    jj = j * UNROLL + u
                    lo = jnp.broadcast_to(jj, (16,)) < n0b
                    r0 = jnp.where(lo, rows[jj, pl.ds(0, 16)],
                                   rows[jj, pl.ds(64, 16)])
                    r1 = jnp.where(lo, rows[jj, pl.ds(16, 16)],
                                   rows[jj, pl.ds(80, 16)])
                    r2 = jnp.where(lo, rows[jj, pl.ds(32, 16)],
                                   rows[jj, pl.ds(96, 16)])
                    r3 = jnp.where(lo, rows[jj, pl.ds(48, 16)],
                                   rows[jj, pl.ds(112, 16)])
                    a0, a1, a2, a3 = a0 + r0, a1 + r1, a2 + r2, a3 + r3
                return a0, a1, a2, a3

            return lax.fori_loop(0, L // UNROLL, body, (zero,) * 4)

        fire(0, rows0, sem0)
        fire(1, rows1, sem1)

        def outer(i, _):
            for phase, (rows, sem) in enumerate(((rows0, sem0), (rows1, sem1))):
                b = 2 * i + phase
                drain(b, rows, sem)
                a0, a1, a2, a3 = accumulate(b, rows)
                nb = b + 2

                @pl.when(nb < BPW)
                def _():
                    fire(nb, rows, sem)

                out_v[b, pl.ds(0, 16)] = a0
                out_v[b, pl.ds(16, 16)] = a1
                out_v[b, pl.ds(32, 16)] = a2
                out_v[b, pl.ds(48, 16)] = a3
            return 0

        lax.fori_loop(0, BPW // 2, outer, 0)
        pltpu.sync_copy(out_v, out_hbm.at[pl.ds(base, BPW)])

    return k(features, table2)


def _tc_head(h, W, b2):
    def body(h_ref, w_ref, b_ref, o_ref):
        hv = jnp.maximum(h_ref[...], 0.0)
        o_ref[...] = lax.dot_general(
            hv, w_ref[...], (((1,), (1,)), ((), ())),
            preferred_element_type=jnp.float32) + b_ref[...]

    return pl.pallas_call(
        body,
        out_shape=jax.ShapeDtypeStruct((B, H), jnp.float32),
    )(h, W, b2)


def kernel(features, table, W, b):
    table2 = _sc_pack_table(jnp.transpose(table), table[VTAIL:, :])
    pooled = _sc_embed_sum(features, table2)
    return _tc_head(pooled, W, b.reshape(1, H))


# speedup vs baseline: 1.9841x; 1.0007x over previous
import functools

import jax
import jax.numpy as jnp
from jax import lax
from jax.experimental import pallas as pl
from jax.experimental.pallas import tpu as pltpu
from jax.experimental.pallas import tpu_sc as plsc

L = 200
B = 4096
H = 64
VP = 500000
HP = 128
NC, NS = 2, 16
NW = NC * NS
BPW = B // NW
CG = BPW // 16
S0, S1 = 96, 104
UNROLL = 8


NSLAB = 3906
VTAIL = 999936


def _sc_pack_table(tableT, tail):
    mesh = plsc.VectorSubcoreMesh(core_axis_name="c", subcore_axis_name="s")

    @functools.partial(
        pl.kernel,
        mesh=mesh,
        out_type=jax.ShapeDtypeStruct((VP, HP), jnp.float32),
        compiler_params=pltpu.CompilerParams(
            use_tc_tiling_on_sc=True, needs_layout_passes=False),
        scratch_types=[
            pltpu.VMEM((H, 256), jnp.float32),
            pltpu.VMEM((H, 256), jnp.float32),
            pltpu.VMEM((128, HP), jnp.float32),
            pltpu.VMEM((128, HP), jnp.float32),
            pltpu.VMEM((64, H), jnp.float32),
            pltpu.SemaphoreType.DMA,
            pltpu.SemaphoreType.DMA,
            pltpu.SemaphoreType.DMA,
            pltpu.SemaphoreType.DMA,
        ],
    )
    def k(tab_hbm, tail_hbm, out_hbm, in0, in1, ob0, ob1, tbuf,
          si0, si1, so0, so1):
        wid = lax.axis_index("s") * NC + lax.axis_index("c")
        iota = lax.iota(jnp.int32, 16)
        hgs = [16 * g + iota for g in range(4)]

        def in_copy(sid, buf, sem):
            return pltpu.make_async_copy(
                tab_hbm.at[:, pl.ds(sid * 256, 256)], buf, sem)

        def out_copy(sid, buf, sem):
            return pltpu.make_async_copy(
                buf, out_hbm.at[pl.ds(sid * 128, 128)], sem)

        def transpose(src, dst):
            def vbody(v0, _):
                for u in range(8):
                    vm = (8 * v0 + u) ^ iota
                    r = lax.shift_right_logical(vm, 1)
                    p64 = (vm & 1) * H
                    for g in range(4):
                        val = plsc.load_gather(src, [hgs[g], vm])
                        plsc.store_scatter(dst, [r, p64 + hgs[g]], val)
                return 0

            lax.fori_loop(0, 32, vbody, 0)

        def sid_of(t):
            return wid + NW * t

        nt = NSLAB // NW + 1

        @pl.when(sid_of(0) < NSLAB)
        def _():
            in_copy(sid_of(0), in0, si0).start()

        @pl.when(sid_of(1) < NSLAB)
        def _():
            in_copy(sid_of(1), in1, si1).start()

        def outer(t, _):
            for phase, (ib, ob, si, so) in enumerate(
                    ((in0, ob0, si0, so0), (in1, ob1, si1, so1))):
                tt = 2 * t + phase
                sid = sid_of(tt)

                @pl.when(sid < NSLAB)
                def _():
                    in_copy(sid, ib, si).wait()

                    @pl.when(tt >= 2)
                    def _():
                        out_copy(sid_of(tt - 2), ob, so).wait()

                    transpose(ib, ob)
                    out_copy(sid, ob, so).start()
                    nsid = sid_of(tt + 2)

                    @pl.when(nsid < NSLAB)
                    def _():
                        in_copy(nsid, ib, si).start()
            return 0

        lax.fori_loop(0, (nt + 1) // 2, outer, 0)

        for tt in (nt - 3, nt - 2, nt - 1):
            ob, so = (ob0, so0) if tt % 2 == 0 else (ob1, so1)

            @pl.when((sid_of(tt) < NSLAB) & (sid_of(tt + 2) >= NSLAB))
            def _():
                out_copy(sid_of(tt), ob, so).wait()

        @pl.when(wid == NW - 1)
        def _():
            pltpu.sync_copy(tail_hbm, tbuf)

            def rbody(r, _):
                for c in range(8):
                    ob0[r, pl.ds(16 * c, 16)] = tbuf[
                        2 * r + c // 4, pl.ds((c % 4) * 16, 16)]
                return 0

            lax.fori_loop(0, 32, rbody, 0)
            pltpu.sync_copy(ob0.at[pl.ds(0, 32)],
                            out_hbm.at[pl.ds(VTAIL // 2, 32)])

    return k(tableT, tail)


def _sc_embed_sum(features, table2):
    mesh = plsc.VectorSubcoreMesh(core_axis_name="c", subcore_axis_name="s")

    @functools.partial(
        pl.kernel,
        mesh=mesh,
        out_type=jax.ShapeDtypeStruct((B, H), jnp.float32),
        compiler_params=pltpu.CompilerParams(
            use_tc_tiling_on_sc=True, needs_layout_passes=False),
        scratch_types=[
            pltpu.VMEM((L, BPW), jnp.int32),
            pltpu.VMEM((BPW * L,), jnp.int32),
            pltpu.VMEM((BPW,), jnp.int32),
            pltpu.VMEM((L, HP), jnp.float32),
            pltpu.VMEM((L, HP), jnp.float32),
            pltpu.VMEM((BPW, H), jnp.float32),
            pltpu.SemaphoreType.DMA,
            pltpu.SemaphoreType.DMA,
        ],
    )
    def k(feat_hbm, table_hbm, out_hbm, idx2d, idx_v, n0_v, rows0, rows1,
          out_v, sem0, sem1):
        wid = lax.axis_index("s") * NC + lax.axis_index("c")
        base = wid * BPW

        pltpu.sync_copy(feat_hbm.at[:, pl.ds(base, BPW)], idx2d)

        lane_base = lax.iota(jnp.int32, 16) * L
        zero16 = jnp.zeros((16,), jnp.int32)
        one16 = jnp.ones((16,), jnp.int32)

        def tbody(l, carry):
            los, his = carry
            nlos, nhis = [], []
            for c in range(CG):
                v = idx2d[l, pl.ds(c * 16, 16)]
                par = v & 1
                v2 = lax.shift_right_logical(v, 1)
                is_hi = par == 1
                pos = lane_base + c * 16 * L + jnp.where(
                    is_hi, (L - 1) - his[c], los[c])
                plsc.store_scatter(idx_v, [pos], v2)
                nlos.append(los[c] + (one16 - par))
                nhis.append(his[c] + par)
            return tuple(nlos), tuple(nhis)

        los, his = lax.fori_loop(
            0, L, tbody, ((zero16,) * CG, (zero16,) * CG))
        for c in range(CG):
            n0_v[pl.ds(c * 16, 16)] = los[c]

        def copies(b, rows, sem):
            off = b * L
            c1 = pltpu.make_async_copy(
                table_hbm.at[idx_v.at[pl.ds(off, S0)]],
                rows.at[pl.ds(0, S0)], sem)
            c2 = pltpu.make_async_copy(
                table_hbm.at[idx_v.at[pl.ds(off + S0, S1)]],
                rows.at[pl.ds(S0, S1)], sem)
            return c1, c2

        def fire(b, rows, sem):
            c1, c2 = copies(b, rows, sem)
            c1.start()
            c2.start()

        def drain(b, rows, sem):
            c1, c2 = copies(b, rows, sem)
            c1.wait()
            c2.wait()

        def accumulate(b, rows):
            zero = jnp.zeros((16,), jnp.float32)
            n0b = plsc.load_gather(n0_v, [jnp.broadcast_to(b, (16,))])

            def body(j, accs):
                a0, a1, a2, a3 = accs
                for u in range(UNROLL):
                    jj = j * UNROLL + u
                    lo = jnp.broadcast_to(jj, (16,)) < n0b
                    r0 = jnp.where(lo, rows[jj, pl.ds(0, 16)],
                                   rows[jj, pl.ds(64, 16)])
                    r1 = jnp.where(lo, rows[jj, pl.ds(16, 16)],
                                   rows[jj, pl.ds(80, 16)])
                    r2 = jnp.where(lo, rows[jj, pl.ds(32, 16)],
                                   rows[jj, pl.ds(96, 16)])
                    r3 = jnp.where(lo, rows[jj, pl.ds(48, 16)],
                                   rows[jj, pl.ds(112, 16)])
                    a0, a1, a2, a3 = a0 + r0, a1 + r1, a2 + r2, a3 + r3
                return a0, a1, a2, a3

            return lax.fori_loop(0, L // UNROLL, body, (zero,) * 4)

        fire(0, rows0, sem0)
        fire(1, rows1, sem1)

        def outer(i, _):
            for phase, (rows, sem) in enumerate(((rows0, sem0), (rows1, sem1))):
                b = 2 * i + phase
                drain(b, rows, sem)
                a0, a1, a2, a3 = accumulate(b, rows)
                nb = b + 2

                @pl.when(nb < BPW)
                def _():
                    fire(nb, rows, sem)

                out_v[b, pl.ds(0, 16)] = a0
                out_v[b, pl.ds(16, 16)] = a1
                out_v[b, pl.ds(32, 16)] = a2
                out_v[b, pl.ds(48, 16)] = a3
            return 0

        lax.fori_loop(0, BPW // 2, outer, 0)
        pltpu.sync_copy(out_v, out_hbm.at[pl.ds(base, BPW)])

    return k(features, table2)


def _tc_head(h, W, b2):
    def body(h_ref, w_ref, b_ref, o_ref):
        hv = jnp.maximum(h_ref[...], 0.0)
        o_ref[...] = lax.dot_general(
            hv, w_ref[...], (((1,), (1,)), ((), ())),
            preferred_element_type=jnp.float32) + b_ref[...]

    return pl.pallas_call(
        body,
        out_shape=jax.ShapeDtypeStruct((B, H), jnp.float32),
    )(h, W, b2)


def kernel(features, table, W, b):
    table2 = _sc_pack_table(jnp.transpose(table), table[VTAIL:, :])
    pooled = _sc_embed_sum(features, table2)
    return _tc_head(pooled, W, b.reshape(1, H))
